# baseline (speedup 1.0000x reference)
import functools

import jax
import jax.numpy as jnp
from jax import lax
from jax.experimental import pallas as pl
from jax.experimental.pallas import tpu as pltpu
from jax.experimental.pallas import tpu_sc as plsc

MODEL_DIM = 64
SCALE = float(MODEL_DIM) ** 0.5

_info = plsc.get_sparse_core_info()
NC, NS, L = _info.num_cores, _info.num_subcores, _info.num_lanes
NW = NC * NS

CHUNK = 128
D_VECS = MODEL_DIM // 16
NBUF = 5
LEAD = 2


def _make_lookup(n_chunks):
    assert n_chunks % NBUF == 0 and n_chunks >= NBUF + LEAD
    n_groups = n_chunks // NBUF
    mesh = plsc.VectorSubcoreMesh(core_axis_name="c", subcore_axis_name="s")

    scratch = [pltpu.VMEM((n_chunks, CHUNK), jnp.int32)]
    scratch += [pltpu.VMEM((CHUNK, MODEL_DIM), jnp.float32) for _ in range(NBUF)]
    scratch += [pltpu.SemaphoreType.DMA for _ in range(2 * NBUF)]

    @functools.partial(
        pl.kernel,
        mesh=mesh,
        compiler_params=pltpu.CompilerParams(use_tc_tiling_on_sc=False),
        out_type=jax.ShapeDtypeStruct((NW, n_chunks, CHUNK, MODEL_DIM), jnp.float32),
        scratch_types=scratch,
    )
    def lookup(idx_hbm, table_hbm, out_hbm, idx_v, *bufs_and_sems):
        bufs = bufs_and_sems[:NBUF]
        gsem = bufs_and_sems[NBUF:2 * NBUF]
        ssem = bufs_and_sems[2 * NBUF:]
        wid = lax.axis_index("s") * NC + lax.axis_index("c")
        pltpu.sync_copy(idx_hbm.at[wid], idx_v)

        for c0 in range(LEAD):
            pltpu.async_copy(table_hbm.at[idx_v.at[c0]], bufs[c0], gsem[c0])

        def group(g, carry):
            for b in range(NBUF):
                c = g * NBUF + b
                r = c + LEAD
                rb = (b + LEAD) % NBUF
                rbuf, rgsem, rssem = bufs[rb], gsem[rb], ssem[rb]

                @pl.when(r < n_chunks)
                def _refill():
                    @pl.when(r >= NBUF)
                    def _wait_store():
                        pltpu.make_async_copy(
                            rbuf, out_hbm.at[wid, 0], rssem
                        ).wait()

                    pltpu.async_copy(table_hbm.at[idx_v.at[r]], rbuf, rgsem)

                buf = bufs[b]
                pltpu.make_async_copy(
                    table_hbm.at[idx_v.at[c]], buf, gsem[b]
                ).wait()

                @plsc.parallel_loop(0, CHUNK, unroll=4)
                def _scale(row):
                    for j in range(D_VECS):
                        buf[row, pl.ds(j * 16, 16)] = (
                            buf[row, pl.ds(j * 16, 16)] * SCALE
                        )

                pltpu.async_copy(buf, out_hbm.at[wid, c], ssem[b])
            return carry

        lax.fori_loop(0, n_groups, group, 0)

        for b in range(NBUF):
            pltpu.make_async_copy(bufs[b], out_hbm.at[wid, 0], ssem[b]).wait()

    return lookup


@jax.jit
def kernel(x, table):
    num_data, seq_len = x.shape
    total = num_data * seq_len
    n_chunks = total // (NW * CHUNK)
    vocab, d = table.shape
    t2 = jnp.reshape(table, (vocab // 2, 2 * d))
    t2 = lax.optimization_barrier(t2)
    tflat = jnp.reshape(t2, (vocab, d))
    idx = x.reshape(NW, n_chunks, CHUNK).astype(jnp.int32)
    out = _make_lookup(n_chunks)(idx, tflat)
    return out.reshape(num_data, seq_len, MODEL_DIM)


# speedup vs baseline: 1.5212x; 1.5212x over previous
import functools

import jax
import jax.numpy as jnp
from jax import lax
from jax.experimental import pallas as pl
from jax.experimental.pallas import tpu as pltpu
from jax.experimental.pallas import tpu_sc as plsc

MODEL_DIM = 64
SCALE = float(MODEL_DIM) ** 0.5

_info = plsc.get_sparse_core_info()
NC, NS, L = _info.num_cores, _info.num_subcores, _info.num_lanes
NW = NC * NS

CHUNK = 128
NBUF = 5
LEAD = 2

TBLOCK = 2048


def _transpose_scale(table_t):
    d, v = table_t.shape
    nblk = pl.cdiv(v, TBLOCK)
    hb = pl.cdiv(nblk, 2)

    def body(in0_ref, in1_ref, out_ref):
        a = jnp.swapaxes(in0_ref[...], 0, 1) * jnp.float32(SCALE)
        b = jnp.swapaxes(in1_ref[...], 0, 1) * jnp.float32(SCALE)
        out_ref[...] = jnp.concatenate([a, b], axis=1)

    return pl.pallas_call(
        body,
        grid=(hb,),
        in_specs=[
            pl.BlockSpec((d, TBLOCK), lambda h: (0, 2 * h)),
            pl.BlockSpec((d, TBLOCK), lambda h: (0, jnp.minimum(2 * h + 1, nblk - 1))),
        ],
        out_specs=pl.BlockSpec((TBLOCK, 2 * d), lambda h: (h, 0)),
        out_shape=jax.ShapeDtypeStruct((hb * TBLOCK, 2 * d), jnp.float32),
    )(table_t, table_t)


def _remap(r):
    b = r // TBLOCK
    w = r % TBLOCK
    return (b // 2) * (2 * TBLOCK) + 2 * w + (b % 2)


def _make_lookup(n_chunks):
    assert n_chunks % NBUF == 0 and n_chunks >= NBUF + LEAD
    n_groups = n_chunks // NBUF
    mesh = plsc.VectorSubcoreMesh(core_axis_name="c", subcore_axis_name="s")

    scratch = [pltpu.VMEM((n_chunks, CHUNK), jnp.int32)]
    scratch += [pltpu.VMEM((CHUNK, MODEL_DIM), jnp.float32) for _ in range(NBUF)]
    scratch += [pltpu.SemaphoreType.DMA for _ in range(2 * NBUF)]

    @functools.partial(
        pl.kernel,
        mesh=mesh,
        compiler_params=pltpu.CompilerParams(use_tc_tiling_on_sc=False),
        out_type=jax.ShapeDtypeStruct((NW, n_chunks, CHUNK, MODEL_DIM), jnp.float32),
        scratch_types=scratch,
    )
    def lookup(idx_hbm, table_hbm, out_hbm, idx_v, *bufs_and_sems):
        bufs = bufs_and_sems[:NBUF]
        gsem = bufs_and_sems[NBUF:2 * NBUF]
        ssem = bufs_and_sems[2 * NBUF:]
        wid = lax.axis_index("s") * NC + lax.axis_index("c")
        pltpu.sync_copy(idx_hbm.at[wid], idx_v)

        for c0 in range(LEAD):
            pltpu.async_copy(table_hbm.at[idx_v.at[c0]], bufs[c0], gsem[c0])

        def group(g, carry):
            for b in range(NBUF):
                c = g * NBUF + b
                r = c + LEAD
                rb = (b + LEAD) % NBUF
                rbuf, rgsem, rssem = bufs[rb], gsem[rb], ssem[rb]

                @pl.when(r < n_chunks)
                def _refill():
                    @pl.when(r >= NBUF)
                    def _wait_store():
                        pltpu.make_async_copy(
                            rbuf, out_hbm.at[wid, 0], rssem
                        ).wait()

                    pltpu.async_copy(table_hbm.at[idx_v.at[r]], rbuf, rgsem)

                buf = bufs[b]
                pltpu.make_async_copy(
                    table_hbm.at[idx_v.at[c]], buf, gsem[b]
                ).wait()
                pltpu.async_copy(buf, out_hbm.at[wid, c], ssem[b])
            return carry

        lax.fori_loop(0, n_groups, group, 0)

        for b in range(NBUF):
            pltpu.make_async_copy(bufs[b], out_hbm.at[wid, 0], ssem[b]).wait()

    return lookup


@jax.jit
def kernel(x, table):
    num_data, seq_len = x.shape
    total = num_data * seq_len
    n_chunks = total // (NW * CHUNK)
    vocab, d = table.shape
    tpacked = _transpose_scale(jnp.swapaxes(table, 0, 1))
    tflat = tpacked.reshape(tpacked.shape[0] * 2, d)
    idx = _remap(x.astype(jnp.int32)).reshape(NW, n_chunks, CHUNK)
    out = _make_lookup(n_chunks)(idx, tflat)
    return out.reshape(num_data, seq_len, MODEL_DIM)


# speedup vs baseline: 1.7511x; 1.1511x over previous
import functools

import jax
import jax.numpy as jnp
from jax import lax
from jax.experimental import pallas as pl
from jax.experimental.pallas import tpu as pltpu
from jax.experimental.pallas import tpu_sc as plsc

MODEL_DIM = 64
SCALE = float(MODEL_DIM) ** 0.5

_info = plsc.get_sparse_core_info()
NC, NS, L = _info.num_cores, _info.num_subcores, _info.num_lanes
NW = NC * NS

CHUNK = 128
NBUF = 5
LEAD = 2

TBLOCK = 4096


def _transpose_scale(table_t):
    d, v = table_t.shape
    nblk = pl.cdiv(v, TBLOCK)
    hb = pl.cdiv(nblk, 2)

    def body(in0_ref, in1_ref, out_ref):
        a = jnp.swapaxes(in0_ref[...], 0, 1) * jnp.float32(SCALE)
        b = jnp.swapaxes(in1_ref[...], 0, 1) * jnp.float32(SCALE)
        out_ref[...] = jnp.concatenate([a, b], axis=1)

    return pl.pallas_call(
        body,
        grid=(hb,),
        in_specs=[
            pl.BlockSpec((d, TBLOCK), lambda h: (0, 2 * h)),
            pl.BlockSpec((d, TBLOCK), lambda h: (0, jnp.minimum(2 * h + 1, nblk - 1))),
        ],
        out_specs=pl.BlockSpec((TBLOCK, 2 * d), lambda h: (h, 0)),
        out_shape=jax.ShapeDtypeStruct((hb * TBLOCK, 2 * d), jnp.float32),
    )(table_t, table_t)


def _remap(r):
    b = r // TBLOCK
    w = r % TBLOCK
    return (b // 2) * (2 * TBLOCK) + 2 * w + (b % 2)


def _make_lookup(n_chunks):
    assert n_chunks % NBUF == 0 and n_chunks >= NBUF + LEAD
    n_groups = n_chunks // NBUF
    mesh = plsc.VectorSubcoreMesh(core_axis_name="c", subcore_axis_name="s")

    scratch = [pltpu.VMEM((n_chunks, CHUNK), jnp.int32)]
    scratch += [pltpu.VMEM((CHUNK, MODEL_DIM), jnp.float32) for _ in range(NBUF)]
    scratch += [pltpu.SemaphoreType.DMA for _ in range(2 * NBUF)]

    @functools.partial(
        pl.kernel,
        mesh=mesh,
        compiler_params=pltpu.CompilerParams(use_tc_tiling_on_sc=False),
        out_type=jax.ShapeDtypeStruct((NW, n_chunks, CHUNK, MODEL_DIM), jnp.float32),
        scratch_types=scratch,
    )
    def lookup(idx_hbm, table_hbm, out_hbm, idx_v, *bufs_and_sems):
        bufs = bufs_and_sems[:NBUF]
        gsem = bufs_and_sems[NBUF:2 * NBUF]
        ssem = bufs_and_sems[2 * NBUF:]
        wid = lax.axis_index("s") * NC + lax.axis_index("c")
        pltpu.sync_copy(idx_hbm.at[wid], idx_v)

        for c0 in range(LEAD):
            pltpu.async_copy(table_hbm.at[idx_v.at[c0]], bufs[c0], gsem[c0])

        def group(g, carry):
            for b in range(NBUF):
                c = g * NBUF + b
                r = c + LEAD
                rb = (b + LEAD) % NBUF
                rbuf, rgsem, rssem = bufs[rb], gsem[rb], ssem[rb]

                @pl.when(r < n_chunks)
                def _refill():
                    @pl.when(r >= NBUF)
                    def _wait_store():
                        pltpu.make_async_copy(
                            rbuf, out_hbm.at[wid, 0], rssem
                        ).wait()

                    pltpu.async_copy(table_hbm.at[idx_v.at[r]], rbuf, rgsem)

                buf = bufs[b]
                pltpu.make_async_copy(
                    table_hbm.at[idx_v.at[c]], buf, gsem[b]
                ).wait()
                pltpu.async_copy(buf, out_hbm.at[wid, c], ssem[b])
            return carry

        lax.fori_loop(0, n_groups, group, 0)

        for b in range(NBUF):
            pltpu.make_async_copy(bufs[b], out_hbm.at[wid, 0], ssem[b]).wait()

    return lookup


@jax.jit
def kernel(x, table):
    num_data, seq_len = x.shape
    total = num_data * seq_len
    n_chunks = total // (NW * CHUNK)
    vocab, d = table.shape
    tpacked = _transpose_scale(jnp.swapaxes(table, 0, 1))
    tflat = tpacked.reshape(tpacked.shape[0] * 2, d)
    idx = _remap(x.astype(jnp.int32)).reshape(NW, n_chunks, CHUNK)
    out = _make_lookup(n_chunks)(idx, tflat)
    return out.reshape(num_data, seq_len, MODEL_DIM)


# speedup vs baseline: 1.8856x; 1.0768x over previous
import functools

import jax
import jax.numpy as jnp
from jax import lax
from jax.experimental import pallas as pl
from jax.experimental.pallas import tpu as pltpu
from jax.experimental.pallas import tpu_sc as plsc

MODEL_DIM = 64
SCALE = float(MODEL_DIM) ** 0.5

_info = plsc.get_sparse_core_info()
NC, NS, L = _info.num_cores, _info.num_subcores, _info.num_lanes
NW = NC * NS

CHUNK = 128
NBUF = 5
LEAD = 2

TBLOCK = 8192


def _transpose_scale(table_t):
    d, v = table_t.shape
    nblk = pl.cdiv(v, TBLOCK)
    hb = pl.cdiv(nblk, 2)

    def body(in0_ref, in1_ref, out_ref):
        a = jnp.swapaxes(in0_ref[...], 0, 1) * jnp.float32(SCALE)
        b = jnp.swapaxes(in1_ref[...], 0, 1) * jnp.float32(SCALE)
        out_ref[...] = jnp.concatenate([a, b], axis=1)

    return pl.pallas_call(
        body,
        grid=(hb,),
        in_specs=[
            pl.BlockSpec((d, TBLOCK), lambda h: (0, 2 * h)),
            pl.BlockSpec((d, TBLOCK), lambda h: (0, jnp.minimum(2 * h + 1, nblk - 1))),
        ],
        out_specs=pl.BlockSpec((TBLOCK, 2 * d), lambda h: (h, 0)),
        out_shape=jax.ShapeDtypeStruct((hb * TBLOCK, 2 * d), jnp.float32),
    )(table_t, table_t)


def _remap(r):
    b = r // TBLOCK
    w = r % TBLOCK
    return (b // 2) * (2 * TBLOCK) + 2 * w + (b % 2)


def _make_lookup(n_chunks):
    assert n_chunks % NBUF == 0 and n_chunks >= NBUF + LEAD
    n_groups = n_chunks // NBUF
    mesh = plsc.VectorSubcoreMesh(core_axis_name="c", subcore_axis_name="s")

    scratch = [pltpu.VMEM((n_chunks, CHUNK), jnp.int32)]
    scratch += [pltpu.VMEM((CHUNK, MODEL_DIM), jnp.float32) for _ in range(NBUF)]
    scratch += [pltpu.SemaphoreType.DMA for _ in range(2 * NBUF)]

    @functools.partial(
        pl.kernel,
        mesh=mesh,
        compiler_params=pltpu.CompilerParams(use_tc_tiling_on_sc=False),
        out_type=jax.ShapeDtypeStruct((NW, n_chunks, CHUNK, MODEL_DIM), jnp.float32),
        scratch_types=scratch,
    )
    def lookup(idx_hbm, table_hbm, out_hbm, idx_v, *bufs_and_sems):
        bufs = bufs_and_sems[:NBUF]
        gsem = bufs_and_sems[NBUF:2 * NBUF]
        ssem = bufs_and_sems[2 * NBUF:]
        wid = lax.axis_index("s") * NC + lax.axis_index("c")
        pltpu.sync_copy(idx_hbm.at[wid], idx_v)

        for c0 in range(LEAD):
            pltpu.async_copy(table_hbm.at[idx_v.at[c0]], bufs[c0], gsem[c0])

        def group(g, carry):
            for b in range(NBUF):
                c = g * NBUF + b
                r = c + LEAD
                rb = (b + LEAD) % NBUF
                rbuf, rgsem, rssem = bufs[rb], gsem[rb], ssem[rb]

                @pl.when(r < n_chunks)
                def _refill():
                    @pl.when(r >= NBUF)
                    def _wait_store():
                        pltpu.make_async_copy(
                            rbuf, out_hbm.at[wid, 0], rssem
                        ).wait()

                    pltpu.async_copy(table_hbm.at[idx_v.at[r]], rbuf, rgsem)

                buf = bufs[b]
                pltpu.make_async_copy(
                    table_hbm.at[idx_v.at[c]], buf, gsem[b]
                ).wait()
                pltpu.async_copy(buf, out_hbm.at[wid, c], ssem[b])
            return carry

        lax.fori_loop(0, n_groups, group, 0)

        for b in range(NBUF):
            pltpu.make_async_copy(bufs[b], out_hbm.at[wid, 0], ssem[b]).wait()

    return lookup


@jax.jit
def kernel(x, table):
    num_data, seq_len = x.shape
    total = num_data * seq_len
    n_chunks = total // (NW * CHUNK)
    vocab, d = table.shape
    tpacked = _transpose_scale(jnp.swapaxes(table, 0, 1))
    tflat = tpacked.reshape(tpacked.shape[0] * 2, d)
    idx = _remap(x.astype(jnp.int32)).reshape(NW, n_chunks, CHUNK)
    out = _make_lookup(n_chunks)(idx, tflat)
    return out.reshape(num_data, seq_len, MODEL_DIM)


# speedup vs baseline: 1.9515x; 1.0350x over previous
import functools

import jax
import jax.numpy as jnp
from jax import lax
from jax.experimental import pallas as pl
from jax.experimental.pallas import tpu as pltpu
from jax.experimental.pallas import tpu_sc as plsc

MODEL_DIM = 64
SCALE = float(MODEL_DIM) ** 0.5

_info = plsc.get_sparse_core_info()
NC, NS, L = _info.num_cores, _info.num_subcores, _info.num_lanes
NW = NC * NS

CHUNK = 128
NBUF = 5
LEAD = 2

TBLOCK = 16384


def _transpose_scale(table_t):
    d, v = table_t.shape
    nblk = pl.cdiv(v, TBLOCK)
    hb = pl.cdiv(nblk, 2)

    def body(in0_ref, in1_ref, out_ref):
        a = jnp.swapaxes(in0_ref[...], 0, 1) * jnp.float32(SCALE)
        b = jnp.swapaxes(in1_ref[...], 0, 1) * jnp.float32(SCALE)
        out_ref[...] = jnp.concatenate([a, b], axis=1)

    return pl.pallas_call(
        body,
        grid=(hb,),
        in_specs=[
            pl.BlockSpec((d, TBLOCK), lambda h: (0, 2 * h)),
            pl.BlockSpec((d, TBLOCK), lambda h: (0, jnp.minimum(2 * h + 1, nblk - 1))),
        ],
        out_specs=pl.BlockSpec((TBLOCK, 2 * d), lambda h: (h, 0)),
        out_shape=jax.ShapeDtypeStruct((hb * TBLOCK, 2 * d), jnp.float32),
    )(table_t, table_t)


def _remap(r):
    b = r // TBLOCK
    w = r % TBLOCK
    return (b // 2) * (2 * TBLOCK) + 2 * w + (b % 2)


def _make_lookup(n_chunks):
    assert n_chunks % NBUF == 0 and n_chunks >= NBUF + LEAD
    n_groups = n_chunks // NBUF
    mesh = plsc.VectorSubcoreMesh(core_axis_name="c", subcore_axis_name="s")

    scratch = [pltpu.VMEM((n_chunks, CHUNK), jnp.int32)]
    scratch += [pltpu.VMEM((CHUNK, MODEL_DIM), jnp.float32) for _ in range(NBUF)]
    scratch += [pltpu.SemaphoreType.DMA for _ in range(2 * NBUF)]

    @functools.partial(
        pl.kernel,
        mesh=mesh,
        compiler_params=pltpu.CompilerParams(use_tc_tiling_on_sc=False),
        out_type=jax.ShapeDtypeStruct((NW, n_chunks, CHUNK, MODEL_DIM), jnp.float32),
        scratch_types=scratch,
    )
    def lookup(idx_hbm, table_hbm, out_hbm, idx_v, *bufs_and_sems):
        bufs = bufs_and_sems[:NBUF]
        gsem = bufs_and_sems[NBUF:2 * NBUF]
        ssem = bufs_and_sems[2 * NBUF:]
        wid = lax.axis_index("s") * NC + lax.axis_index("c")
        pltpu.sync_copy(idx_hbm.at[wid], idx_v)

        for c0 in range(LEAD):
            pltpu.async_copy(table_hbm.at[idx_v.at[c0]], bufs[c0], gsem[c0])

        def group(g, carry):
            for b in range(NBUF):
                c = g * NBUF + b
                r = c + LEAD
                rb = (b + LEAD) % NBUF
                rbuf, rgsem, rssem = bufs[rb], gsem[rb], ssem[rb]

                @pl.when(r < n_chunks)
                def _refill():
                    @pl.when(r >= NBUF)
                    def _wait_store():
                        pltpu.make_async_copy(
                            rbuf, out_hbm.at[wid, 0], rssem
                        ).wait()

                    pltpu.async_copy(table_hbm.at[idx_v.at[r]], rbuf, rgsem)

                buf = bufs[b]
                pltpu.make_async_copy(
                    table_hbm.at[idx_v.at[c]], buf, gsem[b]
                ).wait()
                pltpu.async_copy(buf, out_hbm.at[wid, c], ssem[b])
            return carry

        lax.fori_loop(0, n_groups, group, 0)

        for b in range(NBUF):
            pltpu.make_async_copy(bufs[b], out_hbm.at[wid, 0], ssem[b]).wait()

    return lookup


@jax.jit
def kernel(x, table):
    num_data, seq_len = x.shape
    total = num_data * seq_len
    n_chunks = total // (NW * CHUNK)
    vocab, d = table.shape
    tpacked = _transpose_scale(jnp.swapaxes(table, 0, 1))
    tflat = tpacked.reshape(tpacked.shape[0] * 2, d)
    idx = _remap(x.astype(jnp.int32)).reshape(NW, n_chunks, CHUNK)
    out = _make_lookup(n_chunks)(idx, tflat)
    return out.reshape(num_data, seq_len, MODEL_DIM)
